# baseline (speedup 1.0000x reference)
import jax
import jax.numpy as jnp
from jax import lax
from jax.experimental import pallas as pl
from jax.experimental.pallas import tpu as pltpu
from jax.experimental.pallas import tpu_sc as plsc

_N = 10000
_E = 320000
_D = 128

_NC = 2
_NS = 16
_NW = _NC * _NS
_EPW = _E // _NW
_CHUNK = 125
_NCHUNK = _EPW // _CHUNK
_ISLOT = 20
_NSLOT = _NCHUNK // _ISLOT
_RPT = 632
_RPT_LAST = _N - (_NS - 1) * _RPT


def _pre_body(c_ref, x_ref, w_ref, b_ref, o_ref):
    c = c_ref[0, 0]
    sq = jnp.sqrt(c)
    xb = x_ref[...]
    nrm = jnp.sqrt(jnp.sum(xb * xb, axis=-1, keepdims=True))
    z = sq * nrm
    atanh_z = 0.5 * jnp.log((1.0 + z) / (1.0 - z))
    t = ((2.0 / sq) * atanh_z / nrm) * xb
    o_ref[...] = lax.dot_general(
        t, w_ref[...], (((1,), (1,)), ((), ())),
        preferred_element_type=jnp.float32) + b_ref[...]


def _post_body(c_ref, s_ref, k_ref, o_ref):
    c = c_ref[0, 0]
    sq = jnp.sqrt(c)
    v = (s_ref[0] + s_ref[1]) / jnp.maximum(k_ref[0] + k_ref[1], 1.0)
    nrm = jnp.sqrt(jnp.sum(v * v, axis=-1, keepdims=True))
    o_ref[...] = jnp.tanh(sq * nrm * 0.5) * v / (sq * nrm)


def _sc_body(table, eidx, zrow, zcnt, ones_h,
             sums_out, cnts_out,
             sr0, sr1, dr0, dr1, buf0, buf1, onesv,
             sem0, sem1, csem0, csem1, isem,
             acc, cacc):
    src_ring = (sr0, sr1)
    dst_ring = (dr0, dr1)
    cid = lax.axis_index("c")
    sid = lax.axis_index("s")
    wid = sid * _NC + cid

    pltpu.async_copy(eidx.at[0, wid, 0], src_ring[0], isem)
    pltpu.async_copy(eidx.at[1, wid, 0], dst_ring[0], isem)

    @pl.when(sid < _NS - 1)
    def _():
        pltpu.sync_copy(zrow.at[pl.ds(sid * _RPT, _RPT)],
                        acc.at[pl.ds(sid * _RPT, _RPT)])

    @pl.when(sid == _NS - 1)
    def _():
        pltpu.sync_copy(zrow.at[pl.ds((_NS - 1) * _RPT, _RPT_LAST)],
                        acc.at[pl.ds((_NS - 1) * _RPT, _RPT_LAST)])

    @pl.when(sid == 0)
    def _():
        pltpu.sync_copy(zcnt, cacc)

    pltpu.sync_copy(ones_h, onesv)
    pltpu.make_async_copy(eidx.at[0, wid, 0], src_ring[0], isem).wait()
    pltpu.make_async_copy(eidx.at[1, wid, 0], dst_ring[0], isem).wait()
    plsc.subcore_barrier()

    def wait_count(rc, l, csem):
        pltpu.make_async_copy(onesv, cacc.at[dst_ring[rc].at[l]], csem).wait()

    def step(rc, l, buf, sem, csem, pf, cw):
        pltpu.make_async_copy(table.at[src_ring[rc].at[l]], buf, sem).wait()
        pltpu.sync_copy(buf, acc.at[dst_ring[rc].at[l]], add=True)
        if pf is not None:
            pfc, pfl = pf
            pltpu.async_copy(table.at[src_ring[pfc].at[pfl]], buf, sem)
        if cw is not None:
            cond, cl = cw
            if cond is True:
                wait_count(rc, cl, csem)
            else:
                @pl.when(cond)
                def _():
                    wait_count(rc, cl, csem)
        pltpu.async_copy(onesv, cacc.at[dst_ring[rc].at[l]], csem, add=True)

    pltpu.async_copy(table.at[src_ring[0].at[0]], buf0, sem0)
    pltpu.async_copy(table.at[src_ring[0].at[1]], buf1, sem1)

    for s in range(_NSLOT):
        cur = s % 2
        nxt = (s + 1) % 2
        if s > 0:
            wait_count(nxt, _ISLOT - 2, csem0)
            wait_count(nxt, _ISLOT - 1, csem1)
        if s + 1 < _NSLOT:
            pltpu.async_copy(eidx.at[0, wid, s + 1], src_ring[nxt], isem)
            pltpu.async_copy(eidx.at[1, wid, s + 1], dst_ring[nxt], isem)

        def pair(i, carry, cur=cur):
            l0 = 2 * i
            l1 = l0 + 1
            step(cur, l0, buf0, sem0, csem0, (cur, l0 + 2), (i >= 1, l0 - 2))
            step(cur, l1, buf1, sem1, csem1, (cur, l1 + 2), (i >= 1, l1 - 2))
            return carry

        lax.fori_loop(0, _ISLOT // 2 - 1, pair, 0)

        if s + 1 < _NSLOT:
            pltpu.make_async_copy(eidx.at[0, wid, s + 1], src_ring[nxt], isem).wait()
            pltpu.make_async_copy(eidx.at[1, wid, s + 1], dst_ring[nxt], isem).wait()
            step(cur, _ISLOT - 2, buf0, sem0, csem0, (nxt, 0),
                 (True, _ISLOT - 4))
            step(cur, _ISLOT - 1, buf1, sem1, csem1, (nxt, 1),
                 (True, _ISLOT - 3))
        else:
            step(cur, _ISLOT - 2, buf0, sem0, csem0, None, (True, _ISLOT - 4))
            step(cur, _ISLOT - 1, buf1, sem1, csem1, None, (True, _ISLOT - 3))

    wait_count(1, _ISLOT - 2, csem0)
    wait_count(1, _ISLOT - 1, csem1)

    plsc.subcore_barrier()

    @pl.when(sid < _NS - 1)
    def _():
        pltpu.sync_copy(acc.at[pl.ds(sid * _RPT, _RPT)],
                        sums_out.at[cid, pl.ds(sid * _RPT, _RPT)])

    @pl.when(sid == _NS - 1)
    def _():
        pltpu.sync_copy(acc.at[pl.ds((_NS - 1) * _RPT, _RPT_LAST)],
                        sums_out.at[cid, pl.ds((_NS - 1) * _RPT, _RPT_LAST)])

    @pl.when(sid == 0)
    def _():
        pltpu.sync_copy(cacc, cnts_out.at[cid])


def kernel(x, edge_index, W, b, curvature):
    c2d = jnp.reshape(curvature, (1, 1)).astype(jnp.float32)

    transformed = pl.pallas_call(
        _pre_body,
        grid=(25,),
        in_specs=[
            pl.BlockSpec((1, 1), lambda i: (0, 0), memory_space=pltpu.SMEM),
            pl.BlockSpec((_N // 25, _D), lambda i: (i, 0)),
            pl.BlockSpec((_D, _D), lambda i: (0, 0)),
            pl.BlockSpec((1, _D), lambda i: (0, 0)),
        ],
        out_specs=pl.BlockSpec((_N // 25, _D), lambda i: (i, 0)),
        out_shape=jax.ShapeDtypeStruct((_N, _D), jnp.float32),
    )(c2d, x, W, jnp.reshape(b, (1, _D)))

    eidx = edge_index.reshape(2, _NW, _NSLOT, _ISLOT, _CHUNK)
    zrow = jnp.zeros((_N, _D), jnp.float32)
    zcnt = jnp.zeros((_N,), jnp.float32)
    ones_h = jnp.ones((_CHUNK,), jnp.float32)

    sc = pl.kernel(
        _sc_body,
        out_type=[jax.ShapeDtypeStruct((_NC, _N, _D), jnp.float32),
                  jax.ShapeDtypeStruct((_NC, _N), jnp.float32)],
        mesh=plsc.VectorSubcoreMesh(core_axis_name="c", subcore_axis_name="s"),
        scratch_types=[
            pltpu.VMEM((_ISLOT, _CHUNK), jnp.int32),
            pltpu.VMEM((_ISLOT, _CHUNK), jnp.int32),
            pltpu.VMEM((_ISLOT, _CHUNK), jnp.int32),
            pltpu.VMEM((_ISLOT, _CHUNK), jnp.int32),
            pltpu.VMEM((_CHUNK, _D), jnp.float32),
            pltpu.VMEM((_CHUNK, _D), jnp.float32),
            pltpu.VMEM((_CHUNK,), jnp.float32),
            pltpu.SemaphoreType.DMA,
            pltpu.SemaphoreType.DMA,
            pltpu.SemaphoreType.DMA,
            pltpu.SemaphoreType.DMA,
            pltpu.SemaphoreType.DMA,
            pltpu.VMEM_SHARED((_N, _D), jnp.float32),
            pltpu.VMEM_SHARED((_N,), jnp.float32),
        ],
    )
    ps, pc = sc(transformed, eidx, zrow, zcnt, ones_h)

    cnt = pc.reshape(_NC, _N, 1)
    out = pl.pallas_call(
        _post_body,
        grid=(25,),
        in_specs=[
            pl.BlockSpec((1, 1), lambda i: (0, 0), memory_space=pltpu.SMEM),
            pl.BlockSpec((_NC, _N // 25, _D), lambda i: (0, i, 0)),
            pl.BlockSpec((_NC, _N // 25, 1), lambda i: (0, i, 0)),
        ],
        out_specs=pl.BlockSpec((_N // 25, _D), lambda i: (i, 0)),
        out_shape=jax.ShapeDtypeStruct((_N, _D), jnp.float32),
    )(c2d, ps, cnt)
    return out


# speedup vs baseline: 1.0998x; 1.0998x over previous
import jax
import jax.numpy as jnp
from jax import lax
from jax.experimental import pallas as pl
from jax.experimental.pallas import tpu as pltpu
from jax.experimental.pallas import tpu_sc as plsc

_N = 10000
_E = 320000
_D = 128

_NC = 2
_NS = 16
_NW = _NC * _NS
_EPW = _E // _NW
_CHUNK = 125
_NCHUNK = _EPW // _CHUNK
_ISLOT = 20
_NSLOT = _NCHUNK // _ISLOT
_RPT = 632
_RPT_LAST = _N - (_NS - 1) * _RPT


def _pre_body(c_ref, x_ref, w_ref, b_ref, o_ref):
    c = c_ref[0, 0]
    sq = jnp.sqrt(c)
    xb = x_ref[...]
    nrm = jnp.sqrt(jnp.sum(xb * xb, axis=-1, keepdims=True))
    z = sq * nrm
    atanh_z = 0.5 * jnp.log((1.0 + z) / (1.0 - z))
    t = ((2.0 / sq) * atanh_z / nrm) * xb
    o_ref[...] = lax.dot_general(
        t, w_ref[...], (((1,), (1,)), ((), ())),
        preferred_element_type=jnp.float32) + b_ref[...]


def _post_body(c_ref, s_ref, k_ref, o_ref):
    c = c_ref[0, 0]
    sq = jnp.sqrt(c)
    v = (s_ref[0] + s_ref[1]) / jnp.maximum(k_ref[0] + k_ref[1], 1.0)
    nrm = jnp.sqrt(jnp.sum(v * v, axis=-1, keepdims=True))
    o_ref[...] = jnp.tanh(sq * nrm * 0.5) * v / (sq * nrm)


def _sc_body(table, eidx, zrow, zcnt, ones_h,
             sums_out, cnts_out,
             sr0, sr1, dr0, dr1, buf0, buf1, onesv,
             sem0, sem1, csem0, csem1, isem,
             acc, cacc):
    src_ring = (sr0, sr1)
    dst_ring = (dr0, dr1)
    cid = lax.axis_index("c")
    sid = lax.axis_index("s")
    wid = sid * _NC + cid

    pltpu.async_copy(eidx.at[0, wid, 0], src_ring[0], isem)
    pltpu.async_copy(eidx.at[1, wid, 0], dst_ring[0], isem)

    @pl.when(sid < _NS - 1)
    def _():
        pltpu.sync_copy(zrow.at[pl.ds(sid * _RPT, _RPT)],
                        acc.at[pl.ds(sid * _RPT, _RPT)])

    @pl.when(sid == _NS - 1)
    def _():
        pltpu.sync_copy(zrow.at[pl.ds((_NS - 1) * _RPT, _RPT_LAST)],
                        acc.at[pl.ds((_NS - 1) * _RPT, _RPT_LAST)])

    @pl.when(sid == 0)
    def _():
        pltpu.sync_copy(zcnt, cacc)

    pltpu.sync_copy(ones_h, onesv)
    pltpu.make_async_copy(eidx.at[0, wid, 0], src_ring[0], isem).wait()
    pltpu.make_async_copy(eidx.at[1, wid, 0], dst_ring[0], isem).wait()
    plsc.subcore_barrier()

    def wait_count(rc, l, csem):
        pltpu.make_async_copy(onesv, cacc.at[dst_ring[rc].at[l]], csem).wait()

    def step(rc, l, buf, sem, csem, pf, cw):
        pltpu.make_async_copy(table.at[src_ring[rc].at[l]], buf, sem).wait()
        pltpu.sync_copy(buf, acc.at[dst_ring[rc].at[l]], add=True)
        if pf is not None:
            pfc, pfl = pf
            pltpu.async_copy(table.at[src_ring[pfc].at[pfl]], buf, sem)
        if cw is not None:
            cond, cl = cw
            if cond is True:
                wait_count(rc, cl, csem)
            else:
                @pl.when(cond)
                def _():
                    wait_count(rc, cl, csem)
        pltpu.async_copy(onesv, cacc.at[dst_ring[rc].at[l]], csem, add=True)

    pltpu.async_copy(table.at[src_ring[0].at[0]], buf0, sem0)
    pltpu.async_copy(table.at[src_ring[0].at[1]], buf1, sem1)

    for s in range(_NSLOT):
        cur = s % 2
        nxt = (s + 1) % 2
        if s > 0:
            wait_count(nxt, _ISLOT - 2, csem0)
            wait_count(nxt, _ISLOT - 1, csem1)
        if s + 1 < _NSLOT:
            pltpu.async_copy(eidx.at[0, wid, s + 1], src_ring[nxt], isem)
            pltpu.async_copy(eidx.at[1, wid, s + 1], dst_ring[nxt], isem)

        def pair(i, carry, cur=cur):
            l0 = 2 * i
            l1 = l0 + 1
            step(cur, l0, buf0, sem0, csem0, (cur, l0 + 2), (i >= 1, l0 - 2))
            step(cur, l1, buf1, sem1, csem1, (cur, l1 + 2), (i >= 1, l1 - 2))
            return carry

        lax.fori_loop(0, _ISLOT // 2 - 1, pair, 0)

        if s + 1 < _NSLOT:
            pltpu.make_async_copy(eidx.at[0, wid, s + 1], src_ring[nxt], isem).wait()
            pltpu.make_async_copy(eidx.at[1, wid, s + 1], dst_ring[nxt], isem).wait()
            step(cur, _ISLOT - 2, buf0, sem0, csem0, (nxt, 0),
                 (True, _ISLOT - 4))
            step(cur, _ISLOT - 1, buf1, sem1, csem1, (nxt, 1),
                 (True, _ISLOT - 3))
        else:
            step(cur, _ISLOT - 2, buf0, sem0, csem0, None, (True, _ISLOT - 4))
            step(cur, _ISLOT - 1, buf1, sem1, csem1, None, (True, _ISLOT - 3))

    wait_count(1, _ISLOT - 2, csem0)
    wait_count(1, _ISLOT - 1, csem1)

    plsc.subcore_barrier()

    @pl.when(sid < _NS - 1)
    def _():
        pltpu.sync_copy(acc.at[pl.ds(sid * _RPT, _RPT)],
                        sums_out.at[cid, pl.ds(sid * _RPT, _RPT)])

    @pl.when(sid == _NS - 1)
    def _():
        pltpu.sync_copy(acc.at[pl.ds((_NS - 1) * _RPT, _RPT_LAST)],
                        sums_out.at[cid, pl.ds((_NS - 1) * _RPT, _RPT_LAST)])

    @pl.when(sid == 0)
    def _():
        pltpu.sync_copy(cacc, cnts_out.at[cid])


def kernel(x, edge_index, W, b, curvature):
    c2d = jnp.reshape(curvature, (1, 1)).astype(jnp.float32)

    transformed = pl.pallas_call(
        _pre_body,
        grid=(10,),
        in_specs=[
            pl.BlockSpec((1, 1), lambda i: (0, 0), memory_space=pltpu.SMEM),
            pl.BlockSpec((_N // 10, _D), lambda i: (i, 0)),
            pl.BlockSpec((_D, _D), lambda i: (0, 0)),
            pl.BlockSpec((1, _D), lambda i: (0, 0)),
        ],
        out_specs=pl.BlockSpec((_N // 10, _D), lambda i: (i, 0)),
        out_shape=jax.ShapeDtypeStruct((_N, _D), jnp.float32),
    )(c2d, x, W, jnp.reshape(b, (1, _D)))

    eidx = edge_index.reshape(2, _NW, _NSLOT, _ISLOT, _CHUNK)
    zrow = jnp.zeros((_N, _D), jnp.float32)
    zcnt = jnp.zeros((_N,), jnp.float32)
    ones_h = jnp.ones((_CHUNK,), jnp.float32)

    sc = pl.kernel(
        _sc_body,
        out_type=[jax.ShapeDtypeStruct((_NC, _N, _D), jnp.float32),
                  jax.ShapeDtypeStruct((_NC, _N), jnp.float32)],
        mesh=plsc.VectorSubcoreMesh(core_axis_name="c", subcore_axis_name="s"),
        scratch_types=[
            pltpu.VMEM((_ISLOT, _CHUNK), jnp.int32),
            pltpu.VMEM((_ISLOT, _CHUNK), jnp.int32),
            pltpu.VMEM((_ISLOT, _CHUNK), jnp.int32),
            pltpu.VMEM((_ISLOT, _CHUNK), jnp.int32),
            pltpu.VMEM((_CHUNK, _D), jnp.float32),
            pltpu.VMEM((_CHUNK, _D), jnp.float32),
            pltpu.VMEM((_CHUNK,), jnp.float32),
            pltpu.SemaphoreType.DMA,
            pltpu.SemaphoreType.DMA,
            pltpu.SemaphoreType.DMA,
            pltpu.SemaphoreType.DMA,
            pltpu.SemaphoreType.DMA,
            pltpu.VMEM_SHARED((_N, _D), jnp.float32),
            pltpu.VMEM_SHARED((_N,), jnp.float32),
        ],
    )
    ps, pc = sc(transformed, eidx, zrow, zcnt, ones_h)

    cnt = pc.reshape(_NC, _N, 1)
    out = pl.pallas_call(
        _post_body,
        grid=(10,),
        in_specs=[
            pl.BlockSpec((1, 1), lambda i: (0, 0), memory_space=pltpu.SMEM),
            pl.BlockSpec((_NC, _N // 10, _D), lambda i: (0, i, 0)),
            pl.BlockSpec((_NC, _N // 10, 1), lambda i: (0, i, 0)),
        ],
        out_specs=pl.BlockSpec((_N // 10, _D), lambda i: (i, 0)),
        out_shape=jax.ShapeDtypeStruct((_N, _D), jnp.float32),
    )(c2d, ps, cnt)
    return out
